# baseline (speedup 1.0000x reference)
import functools

import jax
import jax.numpy as jnp
from jax.experimental import pallas as pl
from jax.experimental.pallas import tpu as pltpu

BM = 200
BK = 1024
CACHE_BLKS = 8


def _s1_kernel(x_ref, w1_ref, b1_ref, s1_ref):
    s1_ref[...] = (
        jnp.dot(x_ref[...], w1_ref[...], preferred_element_type=jnp.float32)
        + b1_ref[...]
    )


def _gcn_kernel(s1_ref, adj_ref, w2_ref, b2_ref, out_ref, s2_ref, cache_ref,
                *, num_i, bm, n, c_dim, cache_blks):
    i = pl.program_id(0)
    n_full, edge_w = divmod(n, BK)

    @pl.when(i < num_i)
    def _phase1():
        h = jnp.dot(adj_ref[...], s1_ref[...],
                    preferred_element_type=jnp.float32)
        s2_ref[pl.ds(i * bm, bm), :] = (
            jnp.dot(jnp.maximum(h, 0.0), w2_ref[...],
                    preferred_element_type=jnp.float32)
            + b2_ref[...]
        )

        @pl.when(i < cache_blks)
        def _fill_cache():
            for k in range(n_full):
                cache_ref[i, :, k * BK:(k + 1) * BK] = (
                    adj_ref[:, k * BK:(k + 1) * BK].astype(jnp.bfloat16))
            if edge_w:
                cache_ref[i, :, n_full * BK:n] = (
                    adj_ref[:, n_full * BK:n].astype(jnp.bfloat16))

    @pl.when((i >= num_i) & (i < num_i + cache_blks))
    def _phase2_cached():
        j = i - num_i
        o = jnp.zeros((bm, c_dim), jnp.float32)
        for k in range(n_full):
            o = o + jnp.dot(
                cache_ref[j, :, k * BK:(k + 1) * BK],
                s2_ref[k * BK:(k + 1) * BK, :].astype(jnp.bfloat16),
                preferred_element_type=jnp.float32)
        if edge_w:
            o = o + jnp.dot(
                cache_ref[j, :, n_full * BK:n],
                s2_ref[n_full * BK:n, :].astype(jnp.bfloat16),
                preferred_element_type=jnp.float32)
        m = jnp.max(o, axis=1, keepdims=True)
        lse = jnp.log(jnp.sum(jnp.exp(o - m), axis=1, keepdims=True)) + m
        out_ref[...] = o - lse

    @pl.when(i >= num_i + cache_blks)
    def _phase2():
        o = jnp.dot(adj_ref[...], s2_ref[...],
                    preferred_element_type=jnp.float32)
        m = jnp.max(o, axis=1, keepdims=True)
        lse = jnp.log(jnp.sum(jnp.exp(o - m), axis=1, keepdims=True)) + m
        out_ref[...] = o - lse


@jax.jit
def kernel(x, adj, W1, b1, W2, b2):
    n, f_in = x.shape
    h_dim = W1.shape[1]
    c_dim = W2.shape[1]
    bm = BM if n % BM == 0 else (200 if n % 200 == 0 else 8)
    num_i = n // bm
    cache_blks = min(CACHE_BLKS, num_i - 1)

    b1r = b1.reshape(1, h_dim)
    b2r = b2.reshape(1, c_dim)

    s1 = pl.pallas_call(
        _s1_kernel,
        out_shape=jax.ShapeDtypeStruct((n, h_dim), jnp.float32),
    )(x, W1, b1r)

    def adj_index(i, num_i=num_i, cache_blks=cache_blks):
        j = i - num_i
        return (jnp.where(i < num_i, i,
                          jnp.where(j < cache_blks, num_i - 1, j)), 0)

    return pl.pallas_call(
        functools.partial(_gcn_kernel, num_i=num_i, bm=bm, n=n,
                          c_dim=c_dim, cache_blks=cache_blks),
        grid=(2 * num_i,),
        in_specs=[
            pl.BlockSpec((n, h_dim), lambda i: (0, 0)),
            pl.BlockSpec((bm, n), adj_index),
            pl.BlockSpec((h_dim, c_dim), lambda i: (0, 0)),
            pl.BlockSpec((1, c_dim), lambda i: (0, 0)),
        ],
        out_specs=pl.BlockSpec(
            (bm, c_dim), lambda i, num_i=num_i: (jnp.maximum(i - num_i, 0), 0)
        ),
        out_shape=jax.ShapeDtypeStruct((n, c_dim), jnp.float32),
        scratch_shapes=[
            pltpu.VMEM((n, c_dim), jnp.float32),
            pltpu.VMEM((max(cache_blks, 1), bm, n), jnp.bfloat16),
        ],
        compiler_params=pltpu.CompilerParams(
            dimension_semantics=("arbitrary",),
        ),
    )(s1, adj, W2, b2r)


# speedup vs baseline: 1.0227x; 1.0227x over previous
import functools

import jax
import jax.numpy as jnp
from jax.experimental import pallas as pl
from jax.experimental.pallas import tpu as pltpu

BM = 400
BK = 1024
CACHE_BLKS = 2


def _s1_kernel(x_ref, w1_ref, b1_ref, s1_ref):
    s1_ref[...] = (
        jnp.dot(x_ref[...], w1_ref[...], preferred_element_type=jnp.float32)
        + b1_ref[...]
    )


def _gcn_kernel(s1_ref, adj_ref, w2_ref, b2_ref, out_ref, s2_ref, cache_ref,
                *, num_i, bm, n, c_dim, cache_blks):
    i = pl.program_id(0)
    n_full, edge_w = divmod(n, BK)

    @pl.when(i < num_i)
    def _phase1():
        h = jnp.dot(adj_ref[...], s1_ref[...],
                    preferred_element_type=jnp.float32)
        s2_ref[pl.ds(i * bm, bm), :] = (
            jnp.dot(jnp.maximum(h, 0.0), w2_ref[...],
                    preferred_element_type=jnp.float32)
            + b2_ref[...]
        )

        @pl.when(i < cache_blks)
        def _fill_cache():
            for k in range(n_full):
                cache_ref[i, :, k * BK:(k + 1) * BK] = (
                    adj_ref[:, k * BK:(k + 1) * BK].astype(jnp.bfloat16))
            if edge_w:
                cache_ref[i, :, n_full * BK:n] = (
                    adj_ref[:, n_full * BK:n].astype(jnp.bfloat16))

    @pl.when((i >= num_i) & (i < num_i + cache_blks))
    def _phase2_cached():
        j = i - num_i
        o = jnp.zeros((bm, c_dim), jnp.float32)
        for k in range(n_full):
            o = o + jnp.dot(
                cache_ref[j, :, k * BK:(k + 1) * BK],
                s2_ref[k * BK:(k + 1) * BK, :].astype(jnp.bfloat16),
                preferred_element_type=jnp.float32)
        if edge_w:
            o = o + jnp.dot(
                cache_ref[j, :, n_full * BK:n],
                s2_ref[n_full * BK:n, :].astype(jnp.bfloat16),
                preferred_element_type=jnp.float32)
        m = jnp.max(o, axis=1, keepdims=True)
        lse = jnp.log(jnp.sum(jnp.exp(o - m), axis=1, keepdims=True)) + m
        out_ref[...] = o - lse

    @pl.when(i >= num_i + cache_blks)
    def _phase2():
        o = jnp.dot(adj_ref[...], s2_ref[...],
                    preferred_element_type=jnp.float32)
        m = jnp.max(o, axis=1, keepdims=True)
        lse = jnp.log(jnp.sum(jnp.exp(o - m), axis=1, keepdims=True)) + m
        out_ref[...] = o - lse


@jax.jit
def kernel(x, adj, W1, b1, W2, b2):
    n, f_in = x.shape
    h_dim = W1.shape[1]
    c_dim = W2.shape[1]
    bm = BM if n % BM == 0 else (200 if n % 200 == 0 else 8)
    num_i = n // bm
    cache_blks = min(CACHE_BLKS, num_i - 1)

    b1r = b1.reshape(1, h_dim)
    b2r = b2.reshape(1, c_dim)

    s1 = pl.pallas_call(
        _s1_kernel,
        out_shape=jax.ShapeDtypeStruct((n, h_dim), jnp.float32),
    )(x, W1, b1r)

    def adj_index(i, num_i=num_i, cache_blks=cache_blks):
        j = i - num_i
        return (jnp.where(i < num_i, i,
                          jnp.where(j < cache_blks, num_i - 1, j)), 0)

    return pl.pallas_call(
        functools.partial(_gcn_kernel, num_i=num_i, bm=bm, n=n,
                          c_dim=c_dim, cache_blks=cache_blks),
        grid=(2 * num_i,),
        in_specs=[
            pl.BlockSpec((n, h_dim), lambda i: (0, 0)),
            pl.BlockSpec((bm, n), adj_index),
            pl.BlockSpec((h_dim, c_dim), lambda i: (0, 0)),
            pl.BlockSpec((1, c_dim), lambda i: (0, 0)),
        ],
        out_specs=pl.BlockSpec(
            (bm, c_dim), lambda i, num_i=num_i: (jnp.maximum(i - num_i, 0), 0)
        ),
        out_shape=jax.ShapeDtypeStruct((n, c_dim), jnp.float32),
        scratch_shapes=[
            pltpu.VMEM((n, c_dim), jnp.float32),
            pltpu.VMEM((max(cache_blks, 1), bm, n), jnp.bfloat16),
        ],
        compiler_params=pltpu.CompilerParams(
            dimension_semantics=("arbitrary",),
        ),
    )(s1, adj, W2, b2r)


# speedup vs baseline: 1.0333x; 1.0104x over previous
import functools

import jax
import jax.numpy as jnp
from jax.experimental import pallas as pl
from jax.experimental.pallas import tpu as pltpu

BM = 400
BK = 1024
CACHE_BLKS = 1


def _gcn_kernel(x_ref, adj_ref, w1_ref, b1_ref, w2_ref, b2_ref,
                out_ref, s1_ref, s2_ref, cache_ref,
                *, num_i, bm, n, c_dim, cache_blks):
    i = pl.program_id(0)
    n_full, edge_w = divmod(n, BK)

    @pl.when(i == 0)
    def _init_s1():
        s1_ref[...] = (
            jnp.dot(x_ref[...], w1_ref[...],
                    preferred_element_type=jnp.float32)
            + b1_ref[...]
        )

    @pl.when(i < num_i)
    def _phase1():
        h = jnp.dot(adj_ref[...], s1_ref[...],
                    preferred_element_type=jnp.float32)
        s2_ref[pl.ds(i * bm, bm), :] = (
            jnp.dot(jnp.maximum(h, 0.0), w2_ref[...],
                    preferred_element_type=jnp.float32)
            + b2_ref[...]
        )

        @pl.when(i < cache_blks)
        def _fill_cache():
            for k in range(n_full):
                cache_ref[i, :, k * BK:(k + 1) * BK] = (
                    adj_ref[:, k * BK:(k + 1) * BK].astype(jnp.bfloat16))
            if edge_w:
                cache_ref[i, :, n_full * BK:n] = (
                    adj_ref[:, n_full * BK:n].astype(jnp.bfloat16))

    @pl.when((i >= num_i) & (i < num_i + cache_blks))
    def _phase2_cached():
        j = i - num_i
        o = jnp.zeros((bm, c_dim), jnp.float32)
        for k in range(n_full):
            o = o + jnp.dot(
                cache_ref[j, :, k * BK:(k + 1) * BK],
                s2_ref[k * BK:(k + 1) * BK, :].astype(jnp.bfloat16),
                preferred_element_type=jnp.float32)
        if edge_w:
            o = o + jnp.dot(
                cache_ref[j, :, n_full * BK:n],
                s2_ref[n_full * BK:n, :].astype(jnp.bfloat16),
                preferred_element_type=jnp.float32)
        m = jnp.max(o, axis=1, keepdims=True)
        lse = jnp.log(jnp.sum(jnp.exp(o - m), axis=1, keepdims=True)) + m
        out_ref[...] = o - lse

    @pl.when(i >= num_i + cache_blks)
    def _phase2():
        o = jnp.dot(adj_ref[...], s2_ref[...],
                    preferred_element_type=jnp.float32)
        m = jnp.max(o, axis=1, keepdims=True)
        lse = jnp.log(jnp.sum(jnp.exp(o - m), axis=1, keepdims=True)) + m
        out_ref[...] = o - lse


@jax.jit
def kernel(x, adj, W1, b1, W2, b2):
    n, f_in = x.shape
    h_dim = W1.shape[1]
    c_dim = W2.shape[1]
    bm = BM if n % BM == 0 else (200 if n % 200 == 0 else 8)
    num_i = n // bm
    cache_blks = min(CACHE_BLKS, num_i - 1)

    b1r = b1.reshape(1, h_dim)
    b2r = b2.reshape(1, c_dim)

    def adj_index(i, num_i=num_i, cache_blks=cache_blks):
        j = i - num_i
        return (jnp.where(i < num_i, i,
                          jnp.where(j < cache_blks, num_i - 1, j)), 0)

    return pl.pallas_call(
        functools.partial(_gcn_kernel, num_i=num_i, bm=bm, n=n,
                          c_dim=c_dim, cache_blks=cache_blks),
        grid=(2 * num_i,),
        in_specs=[
            pl.BlockSpec((n, f_in), lambda i: (0, 0)),
            pl.BlockSpec((bm, n), adj_index),
            pl.BlockSpec((f_in, h_dim), lambda i: (0, 0)),
            pl.BlockSpec((1, h_dim), lambda i: (0, 0)),
            pl.BlockSpec((h_dim, c_dim), lambda i: (0, 0)),
            pl.BlockSpec((1, c_dim), lambda i: (0, 0)),
        ],
        out_specs=pl.BlockSpec(
            (bm, c_dim), lambda i, num_i=num_i: (jnp.maximum(i - num_i, 0), 0)
        ),
        out_shape=jax.ShapeDtypeStruct((n, c_dim), jnp.float32),
        scratch_shapes=[
            pltpu.VMEM((n, h_dim), jnp.float32),
            pltpu.VMEM((n, c_dim), jnp.float32),
            pltpu.VMEM((max(cache_blks, 1), bm, n), jnp.bfloat16),
        ],
        compiler_params=pltpu.CompilerParams(
            dimension_semantics=("arbitrary",),
        ),
    )(x, adj, W1, b1r, W2, b2r)
